# baseline (speedup 1.0000x reference)
import functools

import jax
import jax.numpy as jnp
from jax import lax
from jax.experimental import pallas as pl
from jax.experimental.pallas import tpu as pltpu
from jax.experimental.pallas import tpu_sc as plsc

_B, _L = 4096, 200
_WDIM, _PDIM = 64, 5
_FDIM = _WDIM + 2 * _PDIM
_N = _B * _L
_NW = 32
_CHUNK = _B // _NW


def _body(idx_h, wtab_h, p1tab_h, p2tab_h, out_h,
          ibuf0_v, ibuf1_v, wbuf0_v, wbuf1_v, fb0_v, fb1_v,
          p1tab_v, p2tab_v, sem_g, sem_o):
    wid = lax.axis_index("s") * 2 + lax.axis_index("c")
    b0 = wid * _CHUNK
    pltpu.sync_copy(p1tab_h, p1tab_v)
    pltpu.sync_copy(p2tab_h, p2tab_v)

    bufs = ((ibuf0_v, wbuf0_v, fb0_v, 0), (ibuf1_v, wbuf1_v, fb1_v, 1))

    def fire_gather(l, p):
        ibuf_v, wbuf_v, _, k = bufs[p]
        pltpu.sync_copy(idx_h.at[l, :, pl.ds(b0, _CHUNK)], ibuf_v)
        pltpu.async_copy(wtab_h.at[ibuf_v.at[0]], wbuf_v, sem_g.at[k])

    def drain_gather(p):
        ibuf_v, wbuf_v, _, k = bufs[p]
        pltpu.make_async_copy(wtab_h.at[ibuf_v.at[0]], wbuf_v,
                              sem_g.at[k]).wait()

    def assemble(p):
        ibuf_v, wbuf_v, fb_v, _ = bufs[p]
        for j in range(_CHUNK // 16):
            lanes = lax.iota(jnp.int32, 16) + (j * 16)
            for f in range(_WDIM):
                v = plsc.load_gather(wbuf_v,
                                     [lanes, jnp.full((16,), f, jnp.int32)])
                fb_v[f, pl.ds(j * 16, 16)] = v
            f1 = ibuf_v[1, pl.ds(j * 16, 16)] * _PDIM
            f2 = ibuf_v[2, pl.ds(j * 16, 16)] * _PDIM
            for t in range(_PDIM):
                v1 = plsc.load_gather(p1tab_v, [f1 + t])
                fb_v[_WDIM + t, pl.ds(j * 16, 16)] = v1
                v2 = plsc.load_gather(p2tab_v, [f2 + t])
                fb_v[_WDIM + _PDIM + t, pl.ds(j * 16, 16)] = v2

    def wait_out(l, p):
        _, _, fb_v, k = bufs[p]
        pltpu.make_async_copy(
            fb_v, out_h.at[:, pl.ds(l * _B + b0, _CHUNK)], sem_o.at[k]).wait()

    def write_out_async(l, p):
        _, _, fb_v, k = bufs[p]
        pltpu.async_copy(fb_v, out_h.at[:, pl.ds(l * _B + b0, _CHUNK)],
                         sem_o.at[k])

    def write_out_sync(l, p):
        _, _, fb_v, _ = bufs[p]
        pltpu.sync_copy(fb_v, out_h.at[:, pl.ds(l * _B + b0, _CHUNK)])

    fire_gather(0, 0)
    half = _L // 2

    def pair(i, carry):
        l = 2 * i
        drain_gather(0)
        fire_gather(l + 1, 1)

        @pl.when(i >= 1)
        def _():
            wait_out(l, 0)
        assemble(0)

        @pl.when(i < half - 1)
        def _():
            write_out_async(l, 0)

        @pl.when(i == half - 1)
        def _():
            write_out_sync(l, 0)

        drain_gather(1)

        @pl.when(i < half - 1)
        def _():
            fire_gather(l + 2, 0)

        @pl.when(i >= 1)
        def _():
            wait_out(l + 1, 1)
        assemble(1)

        @pl.when(i < half - 1)
        def _():
            write_out_async(l + 1, 1)

        @pl.when(i == half - 1)
        def _():
            write_out_sync(l + 1, 1)

        return carry

    lax.fori_loop(0, half, pair, 0)


def kernel(word, pos1, pos2, word_table, pos1_table, pos2_table):
    idx = jnp.stack(
        [jnp.asarray(word, jnp.int32).T,
         jnp.asarray(pos1, jnp.int32).T,
         jnp.asarray(pos2, jnp.int32).T],
        axis=1)

    mesh = plsc.VectorSubcoreMesh(core_axis_name="c", subcore_axis_name="s")
    f = pl.kernel(
        _body,
        out_type=jax.ShapeDtypeStruct((_FDIM, _N), jnp.float32),
        mesh=mesh,
        compiler_params=pltpu.CompilerParams(
            needs_layout_passes=False, use_tc_tiling_on_sc=False),
        scratch_types=[
            pltpu.VMEM((3, _CHUNK), jnp.int32),
            pltpu.VMEM((3, _CHUNK), jnp.int32),
            pltpu.VMEM((_CHUNK, _WDIM), jnp.float32),
            pltpu.VMEM((_CHUNK, _WDIM), jnp.float32),
            pltpu.VMEM((_FDIM, _CHUNK), jnp.float32),
            pltpu.VMEM((_FDIM, _CHUNK), jnp.float32),
            pltpu.VMEM((2 * _L * _PDIM,), jnp.float32),
            pltpu.VMEM((2 * _L * _PDIM,), jnp.float32),
            pltpu.SemaphoreType.DMA((2,)),
            pltpu.SemaphoreType.DMA((2,)),
        ],
    )
    out_t = f(idx, word_table,
              pos1_table.reshape(-1), pos2_table.reshape(-1))
    return out_t.reshape(_FDIM, _L, _B).transpose(2, 1, 0)


# speedup vs baseline: 1.2192x; 1.2192x over previous
import functools

import jax
import jax.numpy as jnp
from jax import lax
from jax.experimental import pallas as pl
from jax.experimental.pallas import tpu as pltpu
from jax.experimental.pallas import tpu_sc as plsc

_B, _L = 4096, 200
_WDIM, _PDIM = 64, 5
_FDIM = _WDIM + 2 * _PDIM
_N = _B * _L
_NW = 32
_CHUNK = _B // _NW


def _body(idx_h, wtab_h, p1tab_h, p2tab_h, out_h,
          ibuf0_v, ibuf1_v, wbuf0_v, wbuf1_v, fb0_v, fb1_v,
          p1tab_v, p2tab_v, sem_g, sem_o):
    wid = lax.axis_index("s") * 2 + lax.axis_index("c")
    b0 = wid * _CHUNK
    pltpu.sync_copy(p1tab_h, p1tab_v)
    pltpu.sync_copy(p2tab_h, p2tab_v)

    bufs = ((ibuf0_v, wbuf0_v, fb0_v, 0), (ibuf1_v, wbuf1_v, fb1_v, 1))

    def fire_gather(l, p):
        ibuf_v, wbuf_v, _, k = bufs[p]
        pltpu.sync_copy(idx_h.at[l, :, pl.ds(b0, _CHUNK)], ibuf_v)
        pltpu.async_copy(wtab_h.at[ibuf_v.at[0]], wbuf_v, sem_g.at[k])

    def drain_gather(p):
        ibuf_v, wbuf_v, _, k = bufs[p]
        pltpu.make_async_copy(wtab_h.at[ibuf_v.at[0]], wbuf_v,
                              sem_g.at[k]).wait()

    def assemble(p):
        ibuf_v, wbuf_v, fb_v, _ = bufs[p]
        lanes = lax.iota(jnp.int32, 16)
        frows = [lanes + q * 16 for q in range(_WDIM // 16)]

        def row4(i, carry):
            for u in range(4):
                r = i * 4 + u
                cols = jnp.full((16,), 0, jnp.int32) + r
                for q in range(_WDIM // 16):
                    v = wbuf_v[r, pl.ds(q * 16, 16)]
                    plsc.store_scatter(fb_v, [frows[q], cols], v)
            return carry
        lax.fori_loop(0, _CHUNK // 4, row4, 0)

        for j in range(_CHUNK // 16):
            f1 = ibuf_v[1, pl.ds(j * 16, 16)] * _PDIM
            f2 = ibuf_v[2, pl.ds(j * 16, 16)] * _PDIM
            for t in range(_PDIM):
                v1 = plsc.load_gather(p1tab_v, [f1 + t])
                fb_v[_WDIM + t, pl.ds(j * 16, 16)] = v1
                v2 = plsc.load_gather(p2tab_v, [f2 + t])
                fb_v[_WDIM + _PDIM + t, pl.ds(j * 16, 16)] = v2

    def wait_out(l, p):
        _, _, fb_v, k = bufs[p]
        pltpu.make_async_copy(
            fb_v, out_h.at[:, pl.ds(l * _B + b0, _CHUNK)], sem_o.at[k]).wait()

    def write_out_async(l, p):
        _, _, fb_v, k = bufs[p]
        pltpu.async_copy(fb_v, out_h.at[:, pl.ds(l * _B + b0, _CHUNK)],
                         sem_o.at[k])

    def write_out_sync(l, p):
        _, _, fb_v, _ = bufs[p]
        pltpu.sync_copy(fb_v, out_h.at[:, pl.ds(l * _B + b0, _CHUNK)])

    fire_gather(0, 0)
    half = _L // 2

    def pair(i, carry):
        l = 2 * i
        drain_gather(0)
        fire_gather(l + 1, 1)

        @pl.when(i >= 1)
        def _():
            wait_out(l, 0)
        assemble(0)

        @pl.when(i < half - 1)
        def _():
            write_out_async(l, 0)

        @pl.when(i == half - 1)
        def _():
            write_out_sync(l, 0)

        drain_gather(1)

        @pl.when(i < half - 1)
        def _():
            fire_gather(l + 2, 0)

        @pl.when(i >= 1)
        def _():
            wait_out(l + 1, 1)
        assemble(1)

        @pl.when(i < half - 1)
        def _():
            write_out_async(l + 1, 1)

        @pl.when(i == half - 1)
        def _():
            write_out_sync(l + 1, 1)

        return carry

    lax.fori_loop(0, half, pair, 0)


def kernel(word, pos1, pos2, word_table, pos1_table, pos2_table):
    idx = jnp.stack(
        [jnp.asarray(word, jnp.int32).T,
         jnp.asarray(pos1, jnp.int32).T,
         jnp.asarray(pos2, jnp.int32).T],
        axis=1)

    mesh = plsc.VectorSubcoreMesh(core_axis_name="c", subcore_axis_name="s")
    f = pl.kernel(
        _body,
        out_type=jax.ShapeDtypeStruct((_FDIM, _N), jnp.float32),
        mesh=mesh,
        compiler_params=pltpu.CompilerParams(
            needs_layout_passes=False, use_tc_tiling_on_sc=False),
        scratch_types=[
            pltpu.VMEM((3, _CHUNK), jnp.int32),
            pltpu.VMEM((3, _CHUNK), jnp.int32),
            pltpu.VMEM((_CHUNK, _WDIM), jnp.float32),
            pltpu.VMEM((_CHUNK, _WDIM), jnp.float32),
            pltpu.VMEM((_FDIM, _CHUNK), jnp.float32),
            pltpu.VMEM((_FDIM, _CHUNK), jnp.float32),
            pltpu.VMEM((2 * _L * _PDIM,), jnp.float32),
            pltpu.VMEM((2 * _L * _PDIM,), jnp.float32),
            pltpu.SemaphoreType.DMA((2,)),
            pltpu.SemaphoreType.DMA((2,)),
        ],
    )
    out_t = f(idx, word_table,
              pos1_table.reshape(-1), pos2_table.reshape(-1))
    return out_t.reshape(_FDIM, _L, _B).transpose(2, 1, 0)


# speedup vs baseline: 1.6702x; 1.3699x over previous
import functools

import jax
import jax.numpy as jnp
from jax import lax
from jax.experimental import pallas as pl
from jax.experimental.pallas import tpu as pltpu
from jax.experimental.pallas import tpu_sc as plsc

_B, _L = 4096, 200
_WDIM, _PDIM = 64, 5
_FDIM = _WDIM + 2 * _PDIM
_N = _B * _L
_NW = 32
_CHUNK = 128
_SPAN = 2
_ROWS_STEP = _CHUNK * _SPAN
_STEPS = _N // (_NW * _ROWS_STEP)
_G = _N // _ROWS_STEP
_HALF = _STEPS // 2


def _body(idx_h, wtab_h, p1tab_h, p2tab_h, out_h,
          ibuf0_v, ibuf1_v, wbuf0_v, wbuf1_v, pbuf0_v, pbuf1_v,
          p1tab_v, p2tab_v, sem_g, sem_o):
    wid = lax.axis_index("s") * 2 + lax.axis_index("c")
    pltpu.sync_copy(p1tab_h, p1tab_v)
    pltpu.sync_copy(p2tab_h, p2tab_v)

    bufs = ((ibuf0_v, wbuf0_v, pbuf0_v, 0), (ibuf1_v, wbuf1_v, pbuf1_v, 1))

    def fire_gathers(g, b):
        ibuf_v, wbuf_v, _, k = bufs[b]
        pltpu.sync_copy(idx_h.at[g], ibuf_v)
        for c in range(_SPAN):
            pltpu.async_copy(wtab_h.at[ibuf_v.at[0, c]],
                             wbuf_v.at[c], sem_g.at[k])

    def drain_gathers(b):
        ibuf_v, wbuf_v, _, k = bufs[b]
        for c in range(_SPAN):
            pltpu.make_async_copy(wtab_h.at[ibuf_v.at[0, c]],
                                  wbuf_v.at[c], sem_g.at[k]).wait()

    def pos_assemble(b):
        ibuf_v, _, pbuf_v, _ = bufs[b]
        for c in range(_SPAN):
            cs = jnp.full((16,), c, jnp.int32)
            for j in range(_CHUNK // 16):
                rows = lax.iota(jnp.int32, 16) + (j * 16)
                f1 = ibuf_v[1, c, pl.ds(j * 16, 16)] * _PDIM
                f2 = ibuf_v[2, c, pl.ds(j * 16, 16)] * _PDIM
                for t in range(_PDIM):
                    v1 = plsc.load_gather(p1tab_v, [f1 + t])
                    plsc.store_scatter(
                        pbuf_v,
                        [cs, rows, jnp.full((16,), t, jnp.int32)], v1)
                    v2 = plsc.load_gather(p2tab_v, [f2 + t])
                    plsc.store_scatter(
                        pbuf_v,
                        [cs, rows,
                         jnp.full((16,), _PDIM + t, jnp.int32)], v2)

    def wait_out(g, b):
        _, wbuf_v, pbuf_v, k = bufs[b]
        for c in range(_SPAN):
            pltpu.make_async_copy(wbuf_v.at[c], out_h.at[g, c, :, 0:_WDIM],
                                  sem_o.at[k]).wait()
            pltpu.make_async_copy(pbuf_v.at[c],
                                  out_h.at[g, c, :, _WDIM:_FDIM],
                                  sem_o.at[k]).wait()

    def write_out_async(g, b):
        _, wbuf_v, pbuf_v, k = bufs[b]
        for c in range(_SPAN):
            pltpu.async_copy(wbuf_v.at[c], out_h.at[g, c, :, 0:_WDIM],
                             sem_o.at[k])
            pltpu.async_copy(pbuf_v.at[c], out_h.at[g, c, :, _WDIM:_FDIM],
                             sem_o.at[k])

    def write_out_sync(g, b):
        _, wbuf_v, pbuf_v, _ = bufs[b]
        for c in range(_SPAN):
            pltpu.sync_copy(wbuf_v.at[c], out_h.at[g, c, :, 0:_WDIM])
            pltpu.sync_copy(pbuf_v.at[c], out_h.at[g, c, :, _WDIM:_FDIM])

    g0 = wid * _STEPS
    fire_gathers(g0, 0)

    def pair(i, carry):
        g = g0 + 2 * i

        drain_gathers(0)
        fire_gathers(g + 1, 1)

        @pl.when(i >= 1)
        def _():
            wait_out(g, 0)
        pos_assemble(0)

        @pl.when(i < _HALF - 1)
        def _():
            write_out_async(g, 0)

        @pl.when(i == _HALF - 1)
        def _():
            write_out_sync(g, 0)

        drain_gathers(1)

        @pl.when(i < _HALF - 1)
        def _():
            fire_gathers(g + 2, 0)

        @pl.when(i >= 1)
        def _():
            wait_out(g + 1, 1)
        pos_assemble(1)

        @pl.when(i < _HALF - 1)
        def _():
            write_out_async(g + 1, 1)

        @pl.when(i == _HALF - 1)
        def _():
            write_out_sync(g + 1, 1)

        return carry

    lax.fori_loop(0, _HALF, pair, 0)


def kernel(word, pos1, pos2, word_table, pos1_table, pos2_table):
    idx = jnp.stack(
        [jnp.asarray(word, jnp.int32).reshape(_G, _SPAN, _CHUNK),
         jnp.asarray(pos1, jnp.int32).reshape(_G, _SPAN, _CHUNK),
         jnp.asarray(pos2, jnp.int32).reshape(_G, _SPAN, _CHUNK)],
        axis=1)

    mesh = plsc.VectorSubcoreMesh(core_axis_name="c", subcore_axis_name="s")
    f = pl.kernel(
        _body,
        out_type=jax.ShapeDtypeStruct((_G, _SPAN, _CHUNK, _FDIM), jnp.float32),
        mesh=mesh,
        compiler_params=pltpu.CompilerParams(
            needs_layout_passes=False, use_tc_tiling_on_sc=False),
        scratch_types=[
            pltpu.VMEM((3, _SPAN, _CHUNK), jnp.int32),
            pltpu.VMEM((3, _SPAN, _CHUNK), jnp.int32),
            pltpu.VMEM((_SPAN, _CHUNK, _WDIM), jnp.float32),
            pltpu.VMEM((_SPAN, _CHUNK, _WDIM), jnp.float32),
            pltpu.VMEM((_SPAN, _CHUNK, 2 * _PDIM), jnp.float32),
            pltpu.VMEM((_SPAN, _CHUNK, 2 * _PDIM), jnp.float32),
            pltpu.VMEM((2 * _L * _PDIM,), jnp.float32),
            pltpu.VMEM((2 * _L * _PDIM,), jnp.float32),
            pltpu.SemaphoreType.DMA((2,)),
            pltpu.SemaphoreType.DMA((2,)),
        ],
    )
    out = f(idx, word_table,
            pos1_table.reshape(-1), pos2_table.reshape(-1))
    return out.reshape(_B, _L, _FDIM)
